# baseline (speedup 1.0000x reference)
import functools

import jax
import jax.numpy as jnp
from jax import lax
from jax.experimental import pallas as pl
from jax.experimental.pallas import tpu as pltpu

_VMEM_LIMIT = 58 * 1024 * 1024
_TILE_ROWS = 8192
_SLOTS = 6


def _rms_kernel(x_hbm, mean_ref, var_ref, count_ref,
                y_hbm, nmean_ref, nvar_ref, ncount_ref,
                resident, stage, in_sems, out_sems, *,
                n_rows, tiles, tile_rows, slots, epsilon):
    tr = tile_rows

    def dma_in(t, slot):
        return pltpu.make_async_copy(
            x_hbm.at[pl.ds(t * tr, tr)], stage.at[slot], in_sems.at[slot])

    def dma_out(t, slot):
        return pltpu.make_async_copy(
            stage.at[slot], y_hbm.at[pl.ds(t * tr, tr)], out_sems.at[slot])

    for k in range(slots):
        dma_in(k, k).start()

    def stats_body(t, carry):
        s, ss = carry
        slot = jax.lax.rem(t, slots)
        dma_in(t, slot).wait()
        x = stage[slot]
        s = s + jnp.sum(x, axis=0, keepdims=True)
        ss = ss + jnp.sum(x * x, axis=0, keepdims=True)
        resident[t] = x.astype(jnp.bfloat16)

        @pl.when(t + slots < tiles)
        def _():
            dma_in(t + slots, slot).start()

        return s, ss

    zero = jnp.zeros((1, x_hbm.shape[1]), jnp.float32)
    s, ss = jax.lax.fori_loop(0, tiles, stats_body, (zero, zero))

    n = jnp.float32(n_rows)
    bm = s * (1.0 / n)
    bvar = (ss - n * bm * bm) * (1.0 / (n - 1.0))

    mean = mean_ref[...]
    var = var_ref[...]
    cnt = count_ref[0]

    delta = bm - mean
    tot = cnt + n
    new_mean = mean + delta * (n / tot)
    new_var = (var * cnt + bvar * n + delta * delta * (cnt * n / tot)) / tot

    nmean_ref[...] = new_mean
    nvar_ref[...] = new_var
    ncount_ref[0] = tot

    scale = lax.rsqrt(new_var + jnp.float32(epsilon))
    shift = -new_mean * scale

    def norm_body(t, _):
        slot = jax.lax.rem(t, slots)

        @pl.when(t >= slots)
        def _():
            dma_out(t - slots, slot).wait()

        xt = resident[t].astype(jnp.float32)
        stage[slot] = jnp.clip(xt * scale + shift, -5.0, 5.0)
        dma_out(t, slot).start()
        return ()

    jax.lax.fori_loop(0, tiles, norm_body, ())

    def drain_body(t, _):
        dma_out(t, jax.lax.rem(t, slots)).wait()
        return ()

    jax.lax.fori_loop(tiles - slots, tiles, drain_body, ())


def kernel(x, running_mean, running_var, count):
    n, d = x.shape
    tr = _TILE_ROWS
    tiles = n // tr

    rm = running_mean.astype(jnp.float32).reshape(1, d)
    rv = running_var.astype(jnp.float32).reshape(1, d)
    cnt = jnp.asarray(count, jnp.float32).reshape(1)

    body = functools.partial(_rms_kernel, n_rows=n, tiles=tiles,
                             tile_rows=tr, slots=_SLOTS, epsilon=1e-5)

    y, nm, nv, nc = pl.pallas_call(
        body,
        in_specs=[
            pl.BlockSpec(memory_space=pl.ANY),
            pl.BlockSpec(memory_space=pltpu.VMEM),
            pl.BlockSpec(memory_space=pltpu.VMEM),
            pl.BlockSpec(memory_space=pltpu.SMEM),
        ],
        out_specs=(
            pl.BlockSpec(memory_space=pl.ANY),
            pl.BlockSpec(memory_space=pltpu.VMEM),
            pl.BlockSpec(memory_space=pltpu.VMEM),
            pl.BlockSpec(memory_space=pltpu.SMEM),
        ),
        out_shape=(
            jax.ShapeDtypeStruct((n, d), x.dtype),
            jax.ShapeDtypeStruct((1, d), jnp.float32),
            jax.ShapeDtypeStruct((1, d), jnp.float32),
            jax.ShapeDtypeStruct((1,), jnp.float32),
        ),
        scratch_shapes=[
            pltpu.VMEM((tiles, tr, d), jnp.bfloat16),
            pltpu.VMEM((_SLOTS, tr, d), jnp.float32),
            pltpu.SemaphoreType.DMA((_SLOTS,)),
            pltpu.SemaphoreType.DMA((_SLOTS,)),
        ],
        compiler_params=pltpu.CompilerParams(
            vmem_limit_bytes=_VMEM_LIMIT),
    )(x, rm, rv, cnt)

    return y, nm.reshape(d), nv.reshape(d), nc.reshape(())


# speedup vs baseline: 1.0434x; 1.0434x over previous
import functools

import jax
import jax.numpy as jnp
from jax import lax
from jax.experimental import pallas as pl
from jax.experimental.pallas import tpu as pltpu

_VMEM_LIMIT = 58 * 1024 * 1024
_TILE_ROWS = 4096
_IN_SLOTS = 5
_OUT_SLOTS = 5


def _rms_kernel(x_hbm, mean_ref, var_ref, count_ref,
                y_hbm, nmean_ref, nvar_ref, ncount_ref,
                resident, xstage, ystage, in_sems, out_sems, *,
                n_rows, tiles, tile_rows, in_slots, out_slots, epsilon):
    tr = tile_rows

    def dma_in(t, slot):
        return pltpu.make_async_copy(
            x_hbm.at[pl.ds(t * tr, tr)], xstage.at[slot], in_sems.at[slot])

    def dma_out(t, slot):
        return pltpu.make_async_copy(
            ystage.at[slot], y_hbm.at[pl.ds(t * tr, tr)], out_sems.at[slot])

    for k in range(in_slots):
        dma_in(k, k).start()

    def stats_body(t, carry):
        s, ss = carry
        slot = jax.lax.rem(t, in_slots)
        dma_in(t, slot).wait()
        x = xstage[slot]
        s = s + jnp.sum(x, axis=0, keepdims=True)
        ss = ss + jnp.sum(x * x, axis=0, keepdims=True)
        resident[t] = x.astype(jnp.bfloat16)

        @pl.when(t + in_slots < tiles)
        def _():
            dma_in(t + in_slots, slot).start()

        return s, ss

    zero = jnp.zeros((1, x_hbm.shape[1]), jnp.float32)
    s, ss = jax.lax.fori_loop(0, tiles, stats_body, (zero, zero))

    n = jnp.float32(n_rows)
    bm = s * (1.0 / n)
    bvar = (ss - n * bm * bm) * (1.0 / (n - 1.0))

    mean = mean_ref[...]
    var = var_ref[...]
    cnt = count_ref[0]

    delta = bm - mean
    tot = cnt + n
    new_mean = mean + delta * (n / tot)
    new_var = (var * cnt + bvar * n + delta * delta * (cnt * n / tot)) / tot

    nmean_ref[...] = new_mean
    nvar_ref[...] = new_var
    ncount_ref[0] = tot

    scale = lax.rsqrt(new_var + jnp.float32(epsilon))
    shift = -new_mean * scale

    def norm_body(t, _):
        slot = jax.lax.rem(t, out_slots)

        @pl.when(t >= out_slots)
        def _():
            dma_out(t - out_slots, slot).wait()

        xt = resident[t].astype(jnp.float32)
        ystage[slot] = jnp.clip(xt * scale + shift, -5.0, 5.0)
        dma_out(t, slot).start()
        return ()

    jax.lax.fori_loop(0, tiles, norm_body, ())

    def drain_body(t, _):
        dma_out(t, jax.lax.rem(t, out_slots)).wait()
        return ()

    jax.lax.fori_loop(tiles - out_slots, tiles, drain_body, ())


def kernel(x, running_mean, running_var, count):
    n, d = x.shape
    tr = _TILE_ROWS
    tiles = n // tr

    rm = running_mean.astype(jnp.float32).reshape(1, d)
    rv = running_var.astype(jnp.float32).reshape(1, d)
    cnt = jnp.asarray(count, jnp.float32).reshape(1)

    body = functools.partial(_rms_kernel, n_rows=n, tiles=tiles,
                             tile_rows=tr, in_slots=_IN_SLOTS,
                             out_slots=_OUT_SLOTS, epsilon=1e-5)

    y, nm, nv, nc = pl.pallas_call(
        body,
        in_specs=[
            pl.BlockSpec(memory_space=pl.ANY),
            pl.BlockSpec(memory_space=pltpu.VMEM),
            pl.BlockSpec(memory_space=pltpu.VMEM),
            pl.BlockSpec(memory_space=pltpu.SMEM),
        ],
        out_specs=(
            pl.BlockSpec(memory_space=pl.ANY),
            pl.BlockSpec(memory_space=pltpu.VMEM),
            pl.BlockSpec(memory_space=pltpu.VMEM),
            pl.BlockSpec(memory_space=pltpu.SMEM),
        ),
        out_shape=(
            jax.ShapeDtypeStruct((n, d), x.dtype),
            jax.ShapeDtypeStruct((1, d), jnp.float32),
            jax.ShapeDtypeStruct((1, d), jnp.float32),
            jax.ShapeDtypeStruct((1,), jnp.float32),
        ),
        scratch_shapes=[
            pltpu.VMEM((tiles, tr, d), jnp.bfloat16),
            pltpu.VMEM((_IN_SLOTS, tr, d), jnp.float32),
            pltpu.VMEM((_OUT_SLOTS, tr, d), jnp.float32),
            pltpu.SemaphoreType.DMA((_IN_SLOTS,)),
            pltpu.SemaphoreType.DMA((_OUT_SLOTS,)),
        ],
        compiler_params=pltpu.CompilerParams(
            vmem_limit_bytes=_VMEM_LIMIT),
    )(x, rm, rv, cnt)

    return y, nm.reshape(d), nv.reshape(d), nc.reshape(())


# speedup vs baseline: 1.0550x; 1.0111x over previous
import functools

import jax
import jax.numpy as jnp
from jax import lax
from jax.experimental import pallas as pl
from jax.experimental.pallas import tpu as pltpu

_VMEM_LIMIT = 58 * 1024 * 1024
_TILE_ROWS = 2048
_IN_SLOTS = 8
_OUT_SLOTS = 8


def _rms_kernel(x_hbm, mean_ref, var_ref, count_ref,
                y_hbm, nmean_ref, nvar_ref, ncount_ref,
                resident, xstage, ystage, in_sems, out_sems, *,
                n_rows, tiles, tile_rows, in_slots, out_slots, epsilon):
    tr = tile_rows

    def dma_in(t, slot):
        return pltpu.make_async_copy(
            x_hbm.at[pl.ds(t * tr, tr)], xstage.at[slot], in_sems.at[slot])

    def dma_out(t, slot):
        return pltpu.make_async_copy(
            ystage.at[slot], y_hbm.at[pl.ds(t * tr, tr)], out_sems.at[slot])

    for k in range(in_slots):
        dma_in(k, k).start()

    def stats_body(t, carry):
        s, ss = carry
        slot = jax.lax.rem(t, in_slots)
        dma_in(t, slot).wait()
        x = xstage[slot]
        s = s + jnp.sum(x, axis=0, keepdims=True)
        ss = ss + jnp.sum(x * x, axis=0, keepdims=True)
        resident[t] = x.astype(jnp.bfloat16)

        @pl.when(t + in_slots < tiles)
        def _():
            dma_in(t + in_slots, slot).start()

        return s, ss

    zero = jnp.zeros((1, x_hbm.shape[1]), jnp.float32)
    s, ss = jax.lax.fori_loop(0, tiles, stats_body, (zero, zero))

    n = jnp.float32(n_rows)
    bm = s * (1.0 / n)
    bvar = (ss - n * bm * bm) * (1.0 / (n - 1.0))

    mean = mean_ref[...]
    var = var_ref[...]
    cnt = count_ref[0]

    delta = bm - mean
    tot = cnt + n
    new_mean = mean + delta * (n / tot)
    new_var = (var * cnt + bvar * n + delta * delta * (cnt * n / tot)) / tot

    nmean_ref[...] = new_mean
    nvar_ref[...] = new_var
    ncount_ref[0] = tot

    scale = lax.rsqrt(new_var + jnp.float32(epsilon))
    shift = -new_mean * scale

    def norm_body(t, _):
        slot = jax.lax.rem(t, out_slots)

        @pl.when(t >= out_slots)
        def _():
            dma_out(t - out_slots, slot).wait()

        xt = resident[t].astype(jnp.float32)
        ystage[slot] = jnp.clip(xt * scale + shift, -5.0, 5.0)
        dma_out(t, slot).start()
        return ()

    jax.lax.fori_loop(0, tiles, norm_body, ())

    def drain_body(t, _):
        dma_out(t, jax.lax.rem(t, out_slots)).wait()
        return ()

    jax.lax.fori_loop(tiles - out_slots, tiles, drain_body, ())


def kernel(x, running_mean, running_var, count):
    n, d = x.shape
    tr = _TILE_ROWS
    tiles = n // tr

    rm = running_mean.astype(jnp.float32).reshape(1, d)
    rv = running_var.astype(jnp.float32).reshape(1, d)
    cnt = jnp.asarray(count, jnp.float32).reshape(1)

    body = functools.partial(_rms_kernel, n_rows=n, tiles=tiles,
                             tile_rows=tr, in_slots=_IN_SLOTS,
                             out_slots=_OUT_SLOTS, epsilon=1e-5)

    y, nm, nv, nc = pl.pallas_call(
        body,
        in_specs=[
            pl.BlockSpec(memory_space=pl.ANY),
            pl.BlockSpec(memory_space=pltpu.VMEM),
            pl.BlockSpec(memory_space=pltpu.VMEM),
            pl.BlockSpec(memory_space=pltpu.SMEM),
        ],
        out_specs=(
            pl.BlockSpec(memory_space=pl.ANY),
            pl.BlockSpec(memory_space=pltpu.VMEM),
            pl.BlockSpec(memory_space=pltpu.VMEM),
            pl.BlockSpec(memory_space=pltpu.SMEM),
        ),
        out_shape=(
            jax.ShapeDtypeStruct((n, d), x.dtype),
            jax.ShapeDtypeStruct((1, d), jnp.float32),
            jax.ShapeDtypeStruct((1, d), jnp.float32),
            jax.ShapeDtypeStruct((1,), jnp.float32),
        ),
        scratch_shapes=[
            pltpu.VMEM((tiles, tr, d), jnp.bfloat16),
            pltpu.VMEM((_IN_SLOTS, tr, d), jnp.float32),
            pltpu.VMEM((_OUT_SLOTS, tr, d), jnp.float32),
            pltpu.SemaphoreType.DMA((_IN_SLOTS,)),
            pltpu.SemaphoreType.DMA((_OUT_SLOTS,)),
        ],
        compiler_params=pltpu.CompilerParams(
            vmem_limit_bytes=_VMEM_LIMIT),
    )(x, rm, rv, cnt)

    return y, nm.reshape(d), nv.reshape(d), nc.reshape(())


# speedup vs baseline: 1.0710x; 1.0152x over previous
import functools

import jax
import jax.numpy as jnp
from jax import lax
from jax.experimental import pallas as pl
from jax.experimental.pallas import tpu as pltpu

_VMEM_LIMIT = 58 * 1024 * 1024
_TILE_ROWS = 1024
_IN_SLOTS = 10
_OUT_SLOTS = 10


def _rms_kernel(x_hbm, mean_ref, var_ref, count_ref,
                y_hbm, nmean_ref, nvar_ref, ncount_ref,
                resident, xstage, ystage, in_sems, out_sems, *,
                n_rows, tiles, tile_rows, in_slots, out_slots, epsilon):
    tr = tile_rows

    def dma_in(t, slot):
        return pltpu.make_async_copy(
            x_hbm.at[pl.ds(t * tr, tr)], xstage.at[slot], in_sems.at[slot])

    def dma_out(t, slot):
        return pltpu.make_async_copy(
            ystage.at[slot], y_hbm.at[pl.ds(t * tr, tr)], out_sems.at[slot])

    for k in range(in_slots):
        dma_in(k, k).start()

    def stats_body(t, carry):
        s, ss = carry
        slot = jax.lax.rem(t, in_slots)
        dma_in(t, slot).wait()
        x = xstage[slot]
        s = s + jnp.sum(x, axis=0, keepdims=True)
        ss = ss + jnp.sum(x * x, axis=0, keepdims=True)
        resident[t] = x.astype(jnp.bfloat16)

        @pl.when(t + in_slots < tiles)
        def _():
            dma_in(t + in_slots, slot).start()

        return s, ss

    zero = jnp.zeros((1, x_hbm.shape[1]), jnp.float32)
    s, ss = jax.lax.fori_loop(0, tiles, stats_body, (zero, zero))

    n = jnp.float32(n_rows)
    bm = s * (1.0 / n)
    bvar = (ss - n * bm * bm) * (1.0 / (n - 1.0))

    mean = mean_ref[...]
    var = var_ref[...]
    cnt = count_ref[0]

    delta = bm - mean
    tot = cnt + n
    new_mean = mean + delta * (n / tot)
    new_var = (var * cnt + bvar * n + delta * delta * (cnt * n / tot)) / tot

    nmean_ref[...] = new_mean
    nvar_ref[...] = new_var
    ncount_ref[0] = tot

    scale = lax.rsqrt(new_var + jnp.float32(epsilon))
    shift = -new_mean * scale

    def norm_body(t, _):
        slot = jax.lax.rem(t, out_slots)

        @pl.when(t >= out_slots)
        def _():
            dma_out(t - out_slots, slot).wait()

        xt = resident[t].astype(jnp.float32)
        ystage[slot] = jnp.clip(xt * scale + shift, -5.0, 5.0)
        dma_out(t, slot).start()
        return ()

    jax.lax.fori_loop(0, tiles, norm_body, ())

    def drain_body(t, _):
        dma_out(t, jax.lax.rem(t, out_slots)).wait()
        return ()

    jax.lax.fori_loop(tiles - out_slots, tiles, drain_body, ())


def kernel(x, running_mean, running_var, count):
    n, d = x.shape
    tr = _TILE_ROWS
    tiles = n // tr

    rm = running_mean.astype(jnp.float32).reshape(1, d)
    rv = running_var.astype(jnp.float32).reshape(1, d)
    cnt = jnp.asarray(count, jnp.float32).reshape(1)

    body = functools.partial(_rms_kernel, n_rows=n, tiles=tiles,
                             tile_rows=tr, in_slots=_IN_SLOTS,
                             out_slots=_OUT_SLOTS, epsilon=1e-5)

    y, nm, nv, nc = pl.pallas_call(
        body,
        in_specs=[
            pl.BlockSpec(memory_space=pl.ANY),
            pl.BlockSpec(memory_space=pltpu.VMEM),
            pl.BlockSpec(memory_space=pltpu.VMEM),
            pl.BlockSpec(memory_space=pltpu.SMEM),
        ],
        out_specs=(
            pl.BlockSpec(memory_space=pl.ANY),
            pl.BlockSpec(memory_space=pltpu.VMEM),
            pl.BlockSpec(memory_space=pltpu.VMEM),
            pl.BlockSpec(memory_space=pltpu.SMEM),
        ),
        out_shape=(
            jax.ShapeDtypeStruct((n, d), x.dtype),
            jax.ShapeDtypeStruct((1, d), jnp.float32),
            jax.ShapeDtypeStruct((1, d), jnp.float32),
            jax.ShapeDtypeStruct((1,), jnp.float32),
        ),
        scratch_shapes=[
            pltpu.VMEM((tiles, tr, d), jnp.bfloat16),
            pltpu.VMEM((_IN_SLOTS, tr, d), jnp.float32),
            pltpu.VMEM((_OUT_SLOTS, tr, d), jnp.float32),
            pltpu.SemaphoreType.DMA((_IN_SLOTS,)),
            pltpu.SemaphoreType.DMA((_OUT_SLOTS,)),
        ],
        compiler_params=pltpu.CompilerParams(
            vmem_limit_bytes=_VMEM_LIMIT),
    )(x, rm, rv, cnt)

    return y, nm.reshape(d), nv.reshape(d), nc.reshape(())
